# baseline (speedup 1.0000x reference)
import functools

import jax
import jax.numpy as jnp
from jax import lax
from jax.experimental import pallas as pl
from jax.experimental.pallas import tpu as pltpu
from jax.experimental.pallas import tpu_sc as plsc

_B = 8
_N = 2048
_R = 1024
_NB = _N // _R
_NS = 16
_PW = _N // _NS
_PM = 128


def _sc_body(idx_hbm, v_hbm, dval_hbm, hit_hbm,
             idxw, jlw, jlsh, mrg, vloc, dv_loc, ht_loc):
    c = lax.axis_index("c")
    s = lax.axis_index("s")
    jbase = s * _PW

    pltpu.sync_copy(idx_hbm.at[pl.ds(jbase, _PW)], idxw)
    neg1 = jnp.full((16,), -1, jnp.int32)
    lane = lax.iota(jnp.int32, 16)
    for i in range(_N // 16):
        jlw[pl.ds(i * 16, 16)] = neg1
    for ck in range(_PW // 16):
        pv = idxw[pl.ds(ck * 16, 16)]
        jv = jbase + ck * 16 + lane
        for l in range(16):
            plsc.store_scatter(jlw, [pv], jv, mask=lane == l)

    pltpu.sync_copy(jlw, jlsh.at[pl.ds(s * _N, _N)])
    plsc.subcore_barrier()

    @pl.when(s < _R // _PM)
    def _merge():
        mbase = c * _R + s * _PM
        for t in range(_NS):
            pltpu.sync_copy(jlsh.at[pl.ds(t * _N + mbase, _PM)],
                            mrg.at[pl.ds(t * _PM, _PM)])
        pltpu.sync_copy(v_hbm, vloc)

        for ck in range(_PM // 16):
            acc = neg1
            for t in range(_NS):
                acc = jnp.maximum(acc, mrg[pl.ds(t * _PM + ck * 16, 16)])
            hitv = acc >= 0
            jc = jnp.maximum(acc, 0)
            hti = jnp.where(hitv, jnp.full((16,), 1, jnp.int32),
                            jnp.full((16,), 0, jnp.int32))
            for b in range(_B):
                bvec = jnp.full((16,), b, jnp.int32)
                val = plsc.load_gather(vloc, [bvec, jc])
                dv_loc[b, pl.ds(ck * 16, 16)] = val
                ht_loc[b, pl.ds(ck * 16, 16)] = hti
        pltpu.sync_copy(dv_loc, dval_hbm.at[:, pl.ds(mbase, _PM)])
        pltpu.sync_copy(ht_loc, hit_hbm.at[:, pl.ds(mbase, _PM)])


_sc_resolve = pl.kernel(
    _sc_body,
    out_type=(
        jax.ShapeDtypeStruct((_B, _N), jnp.float32),
        jax.ShapeDtypeStruct((_B, _N), jnp.int32),
    ),
    mesh=plsc.VectorSubcoreMesh(core_axis_name="c", subcore_axis_name="s"),
    compiler_params=pltpu.CompilerParams(needs_layout_passes=False),
    scratch_types=[
        pltpu.VMEM((_PW,), jnp.int32),
        pltpu.VMEM((_N,), jnp.int32),
        pltpu.VMEM_SHARED((_NS * _N,), jnp.int32),
        pltpu.VMEM((_NS * _PM,), jnp.int32),
        pltpu.VMEM((_B, _N), jnp.float32),
        pltpu.VMEM((_B, _PM), jnp.float32),
        pltpu.VMEM((_B, _PM), jnp.int32),
    ],
)


_CR = 128
_NCH = _B * _N // _CR
_CPB = _N // _CR
_NBUF = 8


def _tc_body(dval_ref, hit_ref, t_hbm, o_hbm, buf, sem_in, sem_out):
    def in_cp(i, s):
        return pltpu.make_async_copy(
            t_hbm.at[pl.ds(i * _CR, _CR), :], buf.at[s], sem_in.at[s])

    def out_cp(i, s):
        return pltpu.make_async_copy(
            buf.at[s], o_hbm.at[pl.ds(i * _CR, _CR), :], sem_out.at[s])

    for s in range(_NBUF):
        in_cp(s, s).start()

    rowq = lax.broadcasted_iota(jnp.int32, (_CR, _N), 0)
    col = lax.broadcasted_iota(jnp.int32, (_CR, _N), 1)

    def step(i, carry):
        s = lax.rem(i, _NBUF)
        b = lax.div(i, _CPB)
        k = lax.rem(i, _CPB)
        in_cp(i, s).wait()
        dv = dval_ref[pl.ds(b, 1), :]
        ht = hit_ref[pl.ds(b, 1), :]
        mask = (col == k * _CR + rowq) & (ht > 0)
        buf[s, :, :] = jnp.where(mask, dv, buf[s, :, :])
        out_cp(i, s).start()

        @pl.when(i + _NBUF < _NCH)
        def _():
            out_cp(i, s).wait()
            in_cp(i + _NBUF, s).start()

        return carry

    lax.fori_loop(0, _NCH, step, 0)
    for j in range(_NBUF):
        out_cp(_NCH - _NBUF + j, j).wait()


@jax.jit
def kernel(t, idx, v):
    idx32 = idx.astype(jnp.int32)
    dval, hit = _sc_resolve(idx32, v)
    t2 = t.reshape(_B * _N, _N)
    out = pl.pallas_call(
        _tc_body,
        in_specs=[
            pl.BlockSpec(memory_space=pltpu.MemorySpace.VMEM),
            pl.BlockSpec(memory_space=pltpu.MemorySpace.VMEM),
            pl.BlockSpec(memory_space=pltpu.MemorySpace.HBM),
        ],
        out_specs=pl.BlockSpec(memory_space=pltpu.MemorySpace.HBM),
        out_shape=jax.ShapeDtypeStruct((_B * _N, _N), jnp.float32),
        scratch_shapes=[
            pltpu.VMEM((_NBUF, _CR, _N), jnp.float32),
            pltpu.SemaphoreType.DMA((_NBUF,)),
            pltpu.SemaphoreType.DMA((_NBUF,)),
        ],
    )(dval, hit, t2)
    return out.reshape(_B, _N, _N)


# speedup vs baseline: 1.4938x; 1.4938x over previous
import functools

import jax
import jax.numpy as jnp
from jax import lax
from jax.experimental import pallas as pl
from jax.experimental.pallas import tpu as pltpu
from jax.experimental.pallas import tpu_sc as plsc

_B = 8
_N = 2048
_R = 1024
_NB = _N // _R
_NS = 16
_PW = _N // _NS
_PM = 128


def _sc_body(idx_hbm, v_hbm, dval_hbm, hit_hbm,
             idxw, jlw, jlsh, mrg, vloc, dv_loc, ht_loc):
    c = lax.axis_index("c")
    s = lax.axis_index("s")
    jbase = s * _PW

    pltpu.sync_copy(idx_hbm.at[pl.ds(jbase, _PW)], idxw)
    neg1 = jnp.full((16,), -1, jnp.int32)
    lane = lax.iota(jnp.int32, 16)
    for i in range(_N // 16):
        jlw[pl.ds(i * 16, 16)] = neg1
    for ck in range(_PW // 16):
        pv = idxw[pl.ds(ck * 16, 16)]
        jv = jbase + ck * 16 + lane
        for l in range(16):
            plsc.store_scatter(jlw, [pv], jv, mask=lane == l)

    pltpu.sync_copy(jlw, jlsh.at[pl.ds(s * _N, _N)])
    plsc.subcore_barrier()

    @pl.when(s < _R // _PM)
    def _merge():
        mbase = c * _R + s * _PM
        for t in range(_NS):
            pltpu.sync_copy(jlsh.at[pl.ds(t * _N + mbase, _PM)],
                            mrg.at[pl.ds(t * _PM, _PM)])
        pltpu.sync_copy(v_hbm, vloc)

        for ck in range(_PM // 16):
            acc = neg1
            for t in range(_NS):
                acc = jnp.maximum(acc, mrg[pl.ds(t * _PM + ck * 16, 16)])
            hitv = acc >= 0
            jc = jnp.maximum(acc, 0)
            hti = jnp.where(hitv, jnp.full((16,), 1, jnp.int32),
                            jnp.full((16,), 0, jnp.int32))
            for b in range(_B):
                bvec = jnp.full((16,), b, jnp.int32)
                val = plsc.load_gather(vloc, [bvec, jc])
                dv_loc[b, pl.ds(ck * 16, 16)] = val
                ht_loc[b, pl.ds(ck * 16, 16)] = hti
        pltpu.sync_copy(dv_loc, dval_hbm.at[:, pl.ds(mbase, _PM)])
        pltpu.sync_copy(ht_loc, hit_hbm.at[:, pl.ds(mbase, _PM)])


_sc_resolve = pl.kernel(
    _sc_body,
    out_type=(
        jax.ShapeDtypeStruct((_B, _N), jnp.float32),
        jax.ShapeDtypeStruct((_B, _N), jnp.int32),
    ),
    mesh=plsc.VectorSubcoreMesh(core_axis_name="c", subcore_axis_name="s"),
    compiler_params=pltpu.CompilerParams(needs_layout_passes=False),
    scratch_types=[
        pltpu.VMEM((_PW,), jnp.int32),
        pltpu.VMEM((_N,), jnp.int32),
        pltpu.VMEM_SHARED((_NS * _N,), jnp.int32),
        pltpu.VMEM((_NS * _PM,), jnp.int32),
        pltpu.VMEM((_B, _N), jnp.float32),
        pltpu.VMEM((_B, _PM), jnp.float32),
        pltpu.VMEM((_B, _PM), jnp.int32),
    ],
)


_CR = 256
_CPB = _N // _CR
_NBUF = _CPB


def _tc_body(dval_ref, hit_ref, t_hbm, o_hbm, buf, sem_in, sem_out):
    def in_cp(b, k):
        return pltpu.make_async_copy(
            t_hbm.at[pl.ds((b * _CPB + k) * _CR, _CR), :], buf.at[k],
            sem_in.at[k])

    def out_cp(b, k):
        return pltpu.make_async_copy(
            buf.at[k], o_hbm.at[pl.ds((b * _CPB + k) * _CR, _CR), :],
            sem_out.at[k])

    for k in range(_NBUF):
        in_cp(0, k).start()

    diag = (lax.broadcasted_iota(jnp.int32, (_CR, _CR), 0)
            == lax.broadcasted_iota(jnp.int32, (_CR, _CR), 1))

    def step(b, carry):
        for k in range(_CPB):
            in_cp(b, k).wait()
            c0 = k * _CR
            dv = dval_ref[pl.ds(b, 1), c0:c0 + _CR]
            ht = hit_ref[pl.ds(b, 1), c0:c0 + _CR]
            sub = buf[k, :, c0:c0 + _CR]
            buf[k, :, c0:c0 + _CR] = jnp.where(diag & (ht > 0), dv, sub)
            out_cp(b, k).start()

            @pl.when(b < _B - 1)
            def _():
                out_cp(b, k).wait()
                in_cp(b + 1, k).start()

        return carry

    lax.fori_loop(0, _B, step, 0)
    for k in range(_NBUF):
        out_cp(_B - 1, k).wait()


@jax.jit
def kernel(t, idx, v):
    idx32 = idx.astype(jnp.int32)
    dval, hit = _sc_resolve(idx32, v)
    t2 = t.reshape(_B * _N, _N)
    out = pl.pallas_call(
        _tc_body,
        in_specs=[
            pl.BlockSpec(memory_space=pltpu.MemorySpace.VMEM),
            pl.BlockSpec(memory_space=pltpu.MemorySpace.VMEM),
            pl.BlockSpec(memory_space=pltpu.MemorySpace.HBM),
        ],
        out_specs=pl.BlockSpec(memory_space=pltpu.MemorySpace.HBM),
        out_shape=jax.ShapeDtypeStruct((_B * _N, _N), jnp.float32),
        scratch_shapes=[
            pltpu.VMEM((_NBUF, _CR, _N), jnp.float32),
            pltpu.SemaphoreType.DMA((_NBUF,)),
            pltpu.SemaphoreType.DMA((_NBUF,)),
        ],
    )(dval, hit, t2)
    return out.reshape(_B, _N, _N)


# speedup vs baseline: 1.6017x; 1.0722x over previous
import functools

import jax
import jax.numpy as jnp
from jax import lax
from jax.experimental import pallas as pl
from jax.experimental.pallas import tpu as pltpu
from jax.experimental.pallas import tpu_sc as plsc

_B = 8
_N = 2048
_R = 1024
_NB = _N // _R
_NS = 16
_PW = _N // _NS
_PM = 128


def _sc_body(idx_hbm, v_hbm, dval_hbm, hit_hbm,
             idxw, jlw, jlsh, mrg, vloc, dv_loc, ht_loc):
    c = lax.axis_index("c")
    s = lax.axis_index("s")
    jbase = s * _PW

    pltpu.sync_copy(idx_hbm.at[pl.ds(jbase, _PW)], idxw)
    neg1 = jnp.full((16,), -1, jnp.int32)
    lane = lax.iota(jnp.int32, 16)
    for i in range(_N // 16):
        jlw[pl.ds(i * 16, 16)] = neg1
    for ck in range(_PW // 16):
        pv = idxw[pl.ds(ck * 16, 16)]
        jv = jbase + ck * 16 + lane
        for l in range(16):
            plsc.store_scatter(jlw, [pv], jv, mask=lane == l)

    pltpu.sync_copy(jlw, jlsh.at[pl.ds(s * _N, _N)])
    plsc.subcore_barrier()

    @pl.when(s < _R // _PM)
    def _merge():
        mbase = c * _R + s * _PM
        for t in range(_NS):
            pltpu.sync_copy(jlsh.at[pl.ds(t * _N + mbase, _PM)],
                            mrg.at[pl.ds(t * _PM, _PM)])
        pltpu.sync_copy(v_hbm, vloc)

        for ck in range(_PM // 16):
            acc = neg1
            for t in range(_NS):
                acc = jnp.maximum(acc, mrg[pl.ds(t * _PM + ck * 16, 16)])
            hitv = acc >= 0
            jc = jnp.maximum(acc, 0)
            hti = jnp.where(hitv, jnp.full((16,), 1, jnp.int32),
                            jnp.full((16,), 0, jnp.int32))
            for b in range(_B):
                bvec = jnp.full((16,), b, jnp.int32)
                val = plsc.load_gather(vloc, [bvec, jc])
                dv_loc[b, pl.ds(ck * 16, 16)] = val
                ht_loc[b, pl.ds(ck * 16, 16)] = hti
        pltpu.sync_copy(dv_loc, dval_hbm.at[:, pl.ds(mbase, _PM)])
        pltpu.sync_copy(ht_loc, hit_hbm.at[:, pl.ds(mbase, _PM)])


_sc_resolve = pl.kernel(
    _sc_body,
    out_type=(
        jax.ShapeDtypeStruct((_B, _N), jnp.float32),
        jax.ShapeDtypeStruct((_B, _N), jnp.int32),
    ),
    mesh=plsc.VectorSubcoreMesh(core_axis_name="c", subcore_axis_name="s"),
    compiler_params=pltpu.CompilerParams(needs_layout_passes=False),
    scratch_types=[
        pltpu.VMEM((_PW,), jnp.int32),
        pltpu.VMEM((_N,), jnp.int32),
        pltpu.VMEM_SHARED((_NS * _N,), jnp.int32),
        pltpu.VMEM((_NS * _PM,), jnp.int32),
        pltpu.VMEM((_B, _N), jnp.float32),
        pltpu.VMEM((_B, _PM), jnp.float32),
        pltpu.VMEM((_B, _PM), jnp.int32),
    ],
)


_CR = 256
_CPB = _N // _CR
_NSLOT = 2 * _CPB


def _tc_body(dval_ref, hit_ref, t_hbm, o_hbm, buf, sem_in, sem_out):
    def in_cp(b, k, s):
        return pltpu.make_async_copy(
            t_hbm.at[pl.ds((b * _CPB + k) * _CR, _CR), :], buf.at[s],
            sem_in.at[s])

    def out_cp(b, k, s):
        return pltpu.make_async_copy(
            buf.at[s], o_hbm.at[pl.ds((b * _CPB + k) * _CR, _CR), :],
            sem_out.at[s])

    for k in range(_CPB):
        in_cp(0, k, k).start()

    diag = (lax.broadcasted_iota(jnp.int32, (_CR, _CR), 0)
            == lax.broadcasted_iota(jnp.int32, (_CR, _CR), 1))

    def step(b2, carry):
        for e in range(2):
            b = 2 * b2 + e
            for k in range(_CPB):
                s = k + _CPB * e
                so = k + _CPB * (1 - e)
                in_cp(b, k, s).wait()
                c0 = k * _CR
                dv = dval_ref[pl.ds(b, 1), c0:c0 + _CR]
                ht = hit_ref[pl.ds(b, 1), c0:c0 + _CR]
                sub = buf[s, :, c0:c0 + _CR]
                buf[s, :, c0:c0 + _CR] = jnp.where(diag & (ht > 0), dv, sub)
                out_cp(b, k, s).start()

                @pl.when(b >= 1)
                def _():
                    out_cp(b - 1, k, so).wait()

                @pl.when(b + 1 < _B)
                def _():
                    in_cp(b + 1, k, so).start()

        return carry

    lax.fori_loop(0, _B // 2, step, 0)
    for k in range(_CPB):
        out_cp(_B - 1, k, k + _CPB * ((_B - 1) % 2)).wait()


@jax.jit
def kernel(t, idx, v):
    idx32 = idx.astype(jnp.int32)
    dval, hit = _sc_resolve(idx32, v)
    t2 = t.reshape(_B * _N, _N)
    out = pl.pallas_call(
        _tc_body,
        in_specs=[
            pl.BlockSpec(memory_space=pltpu.MemorySpace.VMEM),
            pl.BlockSpec(memory_space=pltpu.MemorySpace.VMEM),
            pl.BlockSpec(memory_space=pltpu.MemorySpace.HBM),
        ],
        out_specs=pl.BlockSpec(memory_space=pltpu.MemorySpace.HBM),
        out_shape=jax.ShapeDtypeStruct((_B * _N, _N), jnp.float32),
        scratch_shapes=[
            pltpu.VMEM((_NSLOT, _CR, _N), jnp.float32),
            pltpu.SemaphoreType.DMA((_NSLOT,)),
            pltpu.SemaphoreType.DMA((_NSLOT,)),
        ],
    )(dval, hit, t2)
    return out.reshape(_B, _N, _N)


# speedup vs baseline: 1.6267x; 1.0156x over previous
import functools

import jax
import jax.numpy as jnp
from jax import lax
from jax.experimental import pallas as pl
from jax.experimental.pallas import tpu as pltpu
from jax.experimental.pallas import tpu_sc as plsc

_B = 8
_N = 2048
_R = 1024
_NB = _N // _R
_NS = 16
_PW = _N // _NS
_PM = 128


def _sc_body(idx_hbm, v_hbm, dval_hbm, hit_hbm,
             idxw, jlw, jlsh, mrg, vloc, dv_loc, ht_loc,
             vsem, wsem1, wsem2):
    c = lax.axis_index("c")
    s = lax.axis_index("s")
    jbase = s * _PW

    @pl.when(s < _R // _PM)
    def _prefetch():
        pltpu.async_copy(v_hbm, vloc, vsem)

    pltpu.sync_copy(idx_hbm.at[pl.ds(jbase, _PW)], idxw)
    neg1 = jnp.full((16,), -1, jnp.int32)
    lane = lax.iota(jnp.int32, 16)
    for i in range(_N // 16):
        jlw[pl.ds(i * 16, 16)] = neg1
    for ck in range(_PW // 16):
        pv = idxw[pl.ds(ck * 16, 16)]
        jv = jbase + ck * 16 + lane
        for l in range(16):
            plsc.store_scatter(jlw, [pv], jv, mask=lane == l)

    pltpu.sync_copy(jlw, jlsh.at[pl.ds(s * _N, _N)])
    plsc.subcore_barrier()

    @pl.when(s < _R // _PM)
    def _merge():
        mbase = c * _R + s * _PM
        for t in range(_NS):
            pltpu.sync_copy(jlsh.at[pl.ds(t * _N + mbase, _PM)],
                            mrg.at[pl.ds(t * _PM, _PM)])
        pltpu.make_async_copy(v_hbm, vloc, vsem).wait()

        for ck in range(_PM // 16):
            acc = neg1
            for t in range(_NS):
                acc = jnp.maximum(acc, mrg[pl.ds(t * _PM + ck * 16, 16)])
            hitv = acc >= 0
            jc = jnp.maximum(acc, 0)
            hti = jnp.where(hitv, jnp.full((16,), 1, jnp.int32),
                            jnp.full((16,), 0, jnp.int32))
            for b in range(_B):
                bvec = jnp.full((16,), b, jnp.int32)
                val = plsc.load_gather(vloc, [bvec, jc])
                dv_loc[b, pl.ds(ck * 16, 16)] = val
                ht_loc[b, pl.ds(ck * 16, 16)] = hti
        h1 = pltpu.async_copy(dv_loc, dval_hbm.at[:, pl.ds(mbase, _PM)], wsem1)
        h2 = pltpu.async_copy(ht_loc, hit_hbm.at[:, pl.ds(mbase, _PM)], wsem2)
        h1.wait()
        h2.wait()


_sc_resolve = pl.kernel(
    _sc_body,
    out_type=(
        jax.ShapeDtypeStruct((_B, _N), jnp.float32),
        jax.ShapeDtypeStruct((_B, _N), jnp.int32),
    ),
    mesh=plsc.VectorSubcoreMesh(core_axis_name="c", subcore_axis_name="s"),
    compiler_params=pltpu.CompilerParams(needs_layout_passes=False),
    scratch_types=[
        pltpu.VMEM((_PW,), jnp.int32),
        pltpu.VMEM((_N,), jnp.int32),
        pltpu.VMEM_SHARED((_NS * _N,), jnp.int32),
        pltpu.VMEM((_NS * _PM,), jnp.int32),
        pltpu.VMEM((_B, _N), jnp.float32),
        pltpu.VMEM((_B, _PM), jnp.float32),
        pltpu.VMEM((_B, _PM), jnp.int32),
        pltpu.SemaphoreType.DMA,
        pltpu.SemaphoreType.DMA,
        pltpu.SemaphoreType.DMA,
    ],
)


_CR = 256
_CPB = _N // _CR
_NSLOT = 2 * _CPB


def _tc_body(dval_ref, hit_ref, t_hbm, o_hbm, buf, sem_in, sem_out):
    def in_cp(b, k, s):
        return pltpu.make_async_copy(
            t_hbm.at[pl.ds((b * _CPB + k) * _CR, _CR), :], buf.at[s],
            sem_in.at[s])

    def out_cp(b, k, s):
        return pltpu.make_async_copy(
            buf.at[s], o_hbm.at[pl.ds((b * _CPB + k) * _CR, _CR), :],
            sem_out.at[s])

    for k in range(_CPB):
        in_cp(0, k, k).start()

    diag = (lax.broadcasted_iota(jnp.int32, (_CR, _CR), 0)
            == lax.broadcasted_iota(jnp.int32, (_CR, _CR), 1))

    def step(b2, carry):
        for e in range(2):
            b = 2 * b2 + e
            for k in range(_CPB):
                s = k + _CPB * e
                so = k + _CPB * (1 - e)
                in_cp(b, k, s).wait()
                c0 = k * _CR
                dv = dval_ref[pl.ds(b, 1), c0:c0 + _CR]
                ht = hit_ref[pl.ds(b, 1), c0:c0 + _CR]
                sub = buf[s, :, c0:c0 + _CR]
                buf[s, :, c0:c0 + _CR] = jnp.where(diag & (ht > 0), dv, sub)
                out_cp(b, k, s).start()

                @pl.when(b >= 1)
                def _():
                    out_cp(b - 1, k, so).wait()

                @pl.when(b + 1 < _B)
                def _():
                    in_cp(b + 1, k, so).start()

        return carry

    lax.fori_loop(0, _B // 2, step, 0)
    for k in range(_CPB):
        out_cp(_B - 1, k, k + _CPB * ((_B - 1) % 2)).wait()


@jax.jit
def kernel(t, idx, v):
    idx32 = idx.astype(jnp.int32)
    dval, hit = _sc_resolve(idx32, v)
    t2 = t.reshape(_B * _N, _N)
    out = pl.pallas_call(
        _tc_body,
        in_specs=[
            pl.BlockSpec(memory_space=pltpu.MemorySpace.VMEM),
            pl.BlockSpec(memory_space=pltpu.MemorySpace.VMEM),
            pl.BlockSpec(memory_space=pltpu.MemorySpace.HBM),
        ],
        out_specs=pl.BlockSpec(memory_space=pltpu.MemorySpace.HBM),
        out_shape=jax.ShapeDtypeStruct((_B * _N, _N), jnp.float32),
        scratch_shapes=[
            pltpu.VMEM((_NSLOT, _CR, _N), jnp.float32),
            pltpu.SemaphoreType.DMA((_NSLOT,)),
            pltpu.SemaphoreType.DMA((_NSLOT,)),
        ],
    )(dval, hit, t2)
    return out.reshape(_B, _N, _N)


# speedup vs baseline: 1.6291x; 1.0015x over previous
import functools

import jax
import jax.numpy as jnp
from jax import lax
from jax.experimental import pallas as pl
from jax.experimental.pallas import tpu as pltpu
from jax.experimental.pallas import tpu_sc as plsc

_B = 8
_N = 2048
_R = 1024
_NB = _N // _R
_NS = 16
_PW = _N // _NS
_PM = 128


def _sc_body(idx_hbm, v_hbm, dval_hbm, hit_hbm,
             idxw, jlw, jlsh, mrg, vloc, dv_loc, ht_loc,
             vsem, wsem1, wsem2):
    c = lax.axis_index("c")
    s = lax.axis_index("s")
    jbase = s * _PW

    @pl.when(s < _R // _PM)
    def _prefetch():
        pltpu.async_copy(v_hbm, vloc, vsem)

    pltpu.sync_copy(idx_hbm.at[pl.ds(jbase, _PW)], idxw)
    neg1 = jnp.full((16,), -1, jnp.int32)
    lane = lax.iota(jnp.int32, 16)
    for i in range(_N // 16):
        jlw[pl.ds(i * 16, 16)] = neg1
    for ck in range(_PW // 16):
        pv = idxw[pl.ds(ck * 16, 16)]
        jv = jbase + ck * 16 + lane
        for l in range(16):
            plsc.store_scatter(jlw, [pv], jv, mask=lane == l)

    pltpu.sync_copy(jlw, jlsh.at[pl.ds(s * _N, _N)])
    plsc.subcore_barrier()

    @pl.when(s < _R // _PM)
    def _merge():
        mbase = c * _R + s * _PM
        for t in range(_NS):
            pltpu.sync_copy(jlsh.at[pl.ds(t * _N + mbase, _PM)],
                            mrg.at[pl.ds(t * _PM, _PM)])
        pltpu.make_async_copy(v_hbm, vloc, vsem).wait()

        for ck in range(_PM // 16):
            acc = neg1
            for t in range(_NS):
                acc = jnp.maximum(acc, mrg[pl.ds(t * _PM + ck * 16, 16)])
            hitv = acc >= 0
            jc = jnp.maximum(acc, 0)
            hti = jnp.where(hitv, jnp.full((16,), 1, jnp.int32),
                            jnp.full((16,), 0, jnp.int32))
            for b in range(_B):
                bvec = jnp.full((16,), b, jnp.int32)
                val = plsc.load_gather(vloc, [bvec, jc])
                dv_loc[b, pl.ds(ck * 16, 16)] = val
                ht_loc[b, pl.ds(ck * 16, 16)] = hti
        h1 = pltpu.async_copy(dv_loc, dval_hbm.at[:, pl.ds(mbase, _PM)], wsem1)
        h2 = pltpu.async_copy(ht_loc, hit_hbm.at[:, pl.ds(mbase, _PM)], wsem2)
        h1.wait()
        h2.wait()


_sc_resolve = pl.kernel(
    _sc_body,
    out_type=(
        jax.ShapeDtypeStruct((_B, _N), jnp.float32),
        jax.ShapeDtypeStruct((_B, _N), jnp.int32),
    ),
    mesh=plsc.VectorSubcoreMesh(core_axis_name="c", subcore_axis_name="s"),
    compiler_params=pltpu.CompilerParams(needs_layout_passes=False),
    scratch_types=[
        pltpu.VMEM((_PW,), jnp.int32),
        pltpu.VMEM((_N,), jnp.int32),
        pltpu.VMEM_SHARED((_NS * _N,), jnp.int32),
        pltpu.VMEM((_NS * _PM,), jnp.int32),
        pltpu.VMEM((_B, _N), jnp.float32),
        pltpu.VMEM((_B, _PM), jnp.float32),
        pltpu.VMEM((_B, _PM), jnp.int32),
        pltpu.SemaphoreType.DMA,
        pltpu.SemaphoreType.DMA,
        pltpu.SemaphoreType.DMA,
    ],
)


_CR = 512
_CPB = _N // _CR
_NSLOT = 2 * _CPB


def _tc_body(dval_ref, hit_ref, t_hbm, o_hbm, buf, sem_in, sem_out):
    def in_cp(b, k, s):
        return pltpu.make_async_copy(
            t_hbm.at[pl.ds((b * _CPB + k) * _CR, _CR), :], buf.at[s],
            sem_in.at[s])

    def out_cp(b, k, s):
        return pltpu.make_async_copy(
            buf.at[s], o_hbm.at[pl.ds((b * _CPB + k) * _CR, _CR), :],
            sem_out.at[s])

    for k in range(_CPB):
        in_cp(0, k, k).start()

    diag = (lax.broadcasted_iota(jnp.int32, (_CR, _CR), 0)
            == lax.broadcasted_iota(jnp.int32, (_CR, _CR), 1))

    def step(b2, carry):
        for e in range(2):
            b = 2 * b2 + e
            for k in range(_CPB):
                s = k + _CPB * e
                so = k + _CPB * (1 - e)
                in_cp(b, k, s).wait()
                c0 = k * _CR
                dv = dval_ref[pl.ds(b, 1), c0:c0 + _CR]
                ht = hit_ref[pl.ds(b, 1), c0:c0 + _CR]
                sub = buf[s, :, c0:c0 + _CR]
                buf[s, :, c0:c0 + _CR] = jnp.where(diag & (ht > 0), dv, sub)
                out_cp(b, k, s).start()

                @pl.when(b >= 1)
                def _():
                    out_cp(b - 1, k, so).wait()

                @pl.when(b + 1 < _B)
                def _():
                    in_cp(b + 1, k, so).start()

        return carry

    lax.fori_loop(0, _B // 2, step, 0)
    for k in range(_CPB):
        out_cp(_B - 1, k, k + _CPB * ((_B - 1) % 2)).wait()


@jax.jit
def kernel(t, idx, v):
    idx32 = idx.astype(jnp.int32)
    dval, hit = _sc_resolve(idx32, v)
    t2 = t.reshape(_B * _N, _N)
    out = pl.pallas_call(
        _tc_body,
        in_specs=[
            pl.BlockSpec(memory_space=pltpu.MemorySpace.VMEM),
            pl.BlockSpec(memory_space=pltpu.MemorySpace.VMEM),
            pl.BlockSpec(memory_space=pltpu.MemorySpace.HBM),
        ],
        out_specs=pl.BlockSpec(memory_space=pltpu.MemorySpace.HBM),
        out_shape=jax.ShapeDtypeStruct((_B * _N, _N), jnp.float32),
        scratch_shapes=[
            pltpu.VMEM((_NSLOT, _CR, _N), jnp.float32),
            pltpu.SemaphoreType.DMA((_NSLOT,)),
            pltpu.SemaphoreType.DMA((_NSLOT,)),
        ],
    )(dval, hit, t2)
    return out.reshape(_B, _N, _N)


# speedup vs baseline: 1.6393x; 1.0062x over previous
import functools

import jax
import jax.numpy as jnp
from jax import lax
from jax.experimental import pallas as pl
from jax.experimental.pallas import tpu as pltpu
from jax.experimental.pallas import tpu_sc as plsc

_B = 8
_N = 2048
_R = 1024
_NB = _N // _R
_NS = 16
_PW = _N // _NS
_PM = 128


def _sc_body(idx_hbm, v_hbm, dval_hbm, hit_hbm,
             idxw, jlw, jlsh, mrg, vloc, dv_loc, ht_loc,
             vsem, wsem1, wsem2):
    c = lax.axis_index("c")
    s = lax.axis_index("s")
    jbase = s * _PW

    @pl.when(s < _R // _PM)
    def _prefetch():
        pltpu.async_copy(v_hbm, vloc, vsem)

    pltpu.sync_copy(idx_hbm.at[pl.ds(jbase, _PW)], idxw)
    neg1 = jnp.full((16,), -1, jnp.int32)
    lane = lax.iota(jnp.int32, 16)
    for i in range(_N // 16):
        jlw[pl.ds(i * 16, 16)] = neg1
    for ck in range(_PW // 16):
        pv = idxw[pl.ds(ck * 16, 16)]
        jv = jbase + ck * 16 + lane
        for l in range(16):
            plsc.store_scatter(jlw, [pv], jv, mask=lane == l)

    pltpu.sync_copy(jlw, jlsh.at[s])
    plsc.subcore_barrier()

    @pl.when(s < _R // _PM)
    def _merge():
        mbase = c * _R + s * _PM
        pltpu.sync_copy(jlsh.at[:, pl.ds(mbase, _PM)], mrg)
        pltpu.make_async_copy(v_hbm, vloc, vsem).wait()

        for ck in range(_PM // 16):
            acc = neg1
            for t in range(_NS):
                acc = jnp.maximum(acc, mrg[t, pl.ds(ck * 16, 16)])
            hitv = acc >= 0
            jc = jnp.maximum(acc, 0)
            hti = jnp.where(hitv, jnp.full((16,), 1, jnp.int32),
                            jnp.full((16,), 0, jnp.int32))
            for b in range(_B):
                bvec = jnp.full((16,), b, jnp.int32)
                val = plsc.load_gather(vloc, [bvec, jc])
                dv_loc[b, pl.ds(ck * 16, 16)] = val
                ht_loc[b, pl.ds(ck * 16, 16)] = hti
        h1 = pltpu.async_copy(dv_loc, dval_hbm.at[:, pl.ds(mbase, _PM)], wsem1)
        h2 = pltpu.async_copy(ht_loc, hit_hbm.at[:, pl.ds(mbase, _PM)], wsem2)
        h1.wait()
        h2.wait()


_sc_resolve = pl.kernel(
    _sc_body,
    out_type=(
        jax.ShapeDtypeStruct((_B, _N), jnp.float32),
        jax.ShapeDtypeStruct((_B, _N), jnp.int32),
    ),
    mesh=plsc.VectorSubcoreMesh(core_axis_name="c", subcore_axis_name="s"),
    compiler_params=pltpu.CompilerParams(needs_layout_passes=False),
    scratch_types=[
        pltpu.VMEM((_PW,), jnp.int32),
        pltpu.VMEM((_N,), jnp.int32),
        pltpu.VMEM_SHARED((_NS, _N), jnp.int32),
        pltpu.VMEM((_NS, _PM), jnp.int32),
        pltpu.VMEM((_B, _N), jnp.float32),
        pltpu.VMEM((_B, _PM), jnp.float32),
        pltpu.VMEM((_B, _PM), jnp.int32),
        pltpu.SemaphoreType.DMA,
        pltpu.SemaphoreType.DMA,
        pltpu.SemaphoreType.DMA,
    ],
)


_CR = 512
_CPB = _N // _CR
_NSLOT = 2 * _CPB


def _tc_body(dval_ref, hit_ref, t_hbm, o_hbm, buf, sem_in, sem_out):
    def in_cp(b, k, s):
        return pltpu.make_async_copy(
            t_hbm.at[pl.ds((b * _CPB + k) * _CR, _CR), :], buf.at[s],
            sem_in.at[s])

    def out_cp(b, k, s):
        return pltpu.make_async_copy(
            buf.at[s], o_hbm.at[pl.ds((b * _CPB + k) * _CR, _CR), :],
            sem_out.at[s])

    for k in range(_CPB):
        in_cp(0, k, k).start()

    diag = (lax.broadcasted_iota(jnp.int32, (_CR, _CR), 0)
            == lax.broadcasted_iota(jnp.int32, (_CR, _CR), 1))

    def step(b2, carry):
        for e in range(2):
            b = 2 * b2 + e
            for k in range(_CPB):
                s = k + _CPB * e
                so = k + _CPB * (1 - e)
                in_cp(b, k, s).wait()
                c0 = k * _CR
                dv = dval_ref[pl.ds(b, 1), c0:c0 + _CR]
                ht = hit_ref[pl.ds(b, 1), c0:c0 + _CR]
                sub = buf[s, :, c0:c0 + _CR]
                buf[s, :, c0:c0 + _CR] = jnp.where(diag & (ht > 0), dv, sub)
                out_cp(b, k, s).start()

                @pl.when(b >= 1)
                def _():
                    out_cp(b - 1, k, so).wait()

                @pl.when(b + 1 < _B)
                def _():
                    in_cp(b + 1, k, so).start()

        return carry

    lax.fori_loop(0, _B // 2, step, 0)
    for k in range(_CPB):
        out_cp(_B - 1, k, k + _CPB * ((_B - 1) % 2)).wait()


@jax.jit
def kernel(t, idx, v):
    idx32 = idx.astype(jnp.int32)
    dval, hit = _sc_resolve(idx32, v)
    t2 = t.reshape(_B * _N, _N)
    out = pl.pallas_call(
        _tc_body,
        in_specs=[
            pl.BlockSpec(memory_space=pltpu.MemorySpace.VMEM),
            pl.BlockSpec(memory_space=pltpu.MemorySpace.VMEM),
            pl.BlockSpec(memory_space=pltpu.MemorySpace.HBM),
        ],
        out_specs=pl.BlockSpec(memory_space=pltpu.MemorySpace.HBM),
        out_shape=jax.ShapeDtypeStruct((_B * _N, _N), jnp.float32),
        scratch_shapes=[
            pltpu.VMEM((_NSLOT, _CR, _N), jnp.float32),
            pltpu.SemaphoreType.DMA((_NSLOT,)),
            pltpu.SemaphoreType.DMA((_NSLOT,)),
        ],
    )(dval, hit, t2)
    return out.reshape(_B, _N, _N)


# speedup vs baseline: 1.6466x; 1.0045x over previous
import functools

import jax
import jax.numpy as jnp
from jax import lax
from jax.experimental import pallas as pl
from jax.experimental.pallas import tpu as pltpu
from jax.experimental.pallas import tpu_sc as plsc

_B = 8
_N = 2048
_R = 1024
_NB = _N // _R
_NS = 16
_PW = _N // _NS
_PM = 128


def _sc_body(idx_hbm, v_hbm, dval_hbm, hit_hbm,
             idxw, jlw, jlsh, mrg, vloc, dv_loc, ht_loc,
             vsem, wsem1, wsem2):
    c = lax.axis_index("c")
    s = lax.axis_index("s")
    jbase = s * _PW

    @pl.when(s < _R // _PM)
    def _prefetch():
        pltpu.async_copy(v_hbm, vloc, vsem)

    pltpu.sync_copy(idx_hbm.at[pl.ds(jbase, _PW)], idxw)
    neg1 = jnp.full((16,), -1, jnp.int32)
    lane = lax.iota(jnp.int32, 16)
    for i in range(_N // 16):
        jlw[pl.ds(i * 16, 16)] = neg1
    for ck in range(_PW // 16):
        pv = idxw[pl.ds(ck * 16, 16)]
        jv = jbase + ck * 16 + lane
        for l in range(16):
            plsc.store_scatter(jlw, [pv], jv, mask=lane == l)

    pltpu.sync_copy(jlw, jlsh.at[s])
    plsc.subcore_barrier()

    @pl.when(s < _R // _PM)
    def _merge():
        mbase = c * _R + s * _PM
        pltpu.sync_copy(jlsh.at[:, pl.ds(mbase, _PM)], mrg)
        pltpu.make_async_copy(v_hbm, vloc, vsem).wait()

        for ck in range(_PM // 16):
            acc = neg1
            for t in range(_NS):
                acc = jnp.maximum(acc, mrg[t, pl.ds(ck * 16, 16)])
            hitv = acc >= 0
            jc = jnp.maximum(acc, 0)
            hti = jnp.where(hitv, jnp.full((16,), 1, jnp.int32),
                            jnp.full((16,), 0, jnp.int32))
            for b in range(_B):
                bvec = jnp.full((16,), b, jnp.int32)
                val = plsc.load_gather(vloc, [bvec, jc])
                dv_loc[b, pl.ds(ck * 16, 16)] = val
                ht_loc[b, pl.ds(ck * 16, 16)] = hti
        h1 = pltpu.async_copy(dv_loc, dval_hbm.at[:, pl.ds(mbase, _PM)], wsem1)
        h2 = pltpu.async_copy(ht_loc, hit_hbm.at[:, pl.ds(mbase, _PM)], wsem2)
        h1.wait()
        h2.wait()


_sc_resolve = pl.kernel(
    _sc_body,
    out_type=(
        jax.ShapeDtypeStruct((_B, _N), jnp.float32),
        jax.ShapeDtypeStruct((_B, _N), jnp.int32),
    ),
    mesh=plsc.VectorSubcoreMesh(core_axis_name="c", subcore_axis_name="s"),
    compiler_params=pltpu.CompilerParams(needs_layout_passes=False),
    scratch_types=[
        pltpu.VMEM((_PW,), jnp.int32),
        pltpu.VMEM((_N,), jnp.int32),
        pltpu.VMEM_SHARED((_NS, _N), jnp.int32),
        pltpu.VMEM((_NS, _PM), jnp.int32),
        pltpu.VMEM((_B, _N), jnp.float32),
        pltpu.VMEM((_B, _PM), jnp.float32),
        pltpu.VMEM((_B, _PM), jnp.int32),
        pltpu.SemaphoreType.DMA,
        pltpu.SemaphoreType.DMA,
        pltpu.SemaphoreType.DMA,
    ],
)


_CR = 1024
_CPB = _N // _CR
_NSLOT = 2 * _CPB


def _tc_body(dval_ref, hit_ref, t_hbm, o_hbm, buf, sem_in, sem_out):
    def in_cp(b, k, s):
        return pltpu.make_async_copy(
            t_hbm.at[pl.ds((b * _CPB + k) * _CR, _CR), :], buf.at[s],
            sem_in.at[s])

    def out_cp(b, k, s):
        return pltpu.make_async_copy(
            buf.at[s], o_hbm.at[pl.ds((b * _CPB + k) * _CR, _CR), :],
            sem_out.at[s])

    for k in range(_CPB):
        in_cp(0, k, k).start()

    diag = (lax.broadcasted_iota(jnp.int32, (_CR, _CR), 0)
            == lax.broadcasted_iota(jnp.int32, (_CR, _CR), 1))

    def step(b2, carry):
        for e in range(2):
            b = 2 * b2 + e
            for k in range(_CPB):
                s = k + _CPB * e
                so = k + _CPB * (1 - e)
                in_cp(b, k, s).wait()
                c0 = k * _CR
                dv = dval_ref[pl.ds(b, 1), c0:c0 + _CR]
                ht = hit_ref[pl.ds(b, 1), c0:c0 + _CR]
                sub = buf[s, :, c0:c0 + _CR]
                buf[s, :, c0:c0 + _CR] = jnp.where(diag & (ht > 0), dv, sub)
                out_cp(b, k, s).start()

                @pl.when(b >= 1)
                def _():
                    out_cp(b - 1, k, so).wait()

                @pl.when(b + 1 < _B)
                def _():
                    in_cp(b + 1, k, so).start()

        return carry

    lax.fori_loop(0, _B // 2, step, 0)
    for k in range(_CPB):
        out_cp(_B - 1, k, k + _CPB * ((_B - 1) % 2)).wait()


@jax.jit
def kernel(t, idx, v):
    idx32 = idx.astype(jnp.int32)
    dval, hit = _sc_resolve(idx32, v)
    t2 = t.reshape(_B * _N, _N)
    out = pl.pallas_call(
        _tc_body,
        in_specs=[
            pl.BlockSpec(memory_space=pltpu.MemorySpace.VMEM),
            pl.BlockSpec(memory_space=pltpu.MemorySpace.VMEM),
            pl.BlockSpec(memory_space=pltpu.MemorySpace.HBM),
        ],
        out_specs=pl.BlockSpec(memory_space=pltpu.MemorySpace.HBM),
        out_shape=jax.ShapeDtypeStruct((_B * _N, _N), jnp.float32),
        scratch_shapes=[
            pltpu.VMEM((_NSLOT, _CR, _N), jnp.float32),
            pltpu.SemaphoreType.DMA((_NSLOT,)),
            pltpu.SemaphoreType.DMA((_NSLOT,)),
        ],
    )(dval, hit, t2)
    return out.reshape(_B, _N, _N)
